# baseline (speedup 1.0000x reference)
import jax
import jax.numpy as jnp
from jax.experimental import pallas as pl
from jax.experimental.pallas import tpu as pltpu

_N = 10000
_D = 128
_H = 128
_C = 64
_BM = 200
_G = _N // _BM


def _jknet_body(x_ref, adj_ref, W1_ref, b1_ref, W2_ref, b2_ref, W3_ref,
                b3_ref, Wf_ref, bf_ref, logp_ref, p_ref,
                h1_ref, h2_ref, sup_ref):
    l = pl.program_id(0)
    i = pl.program_id(1)

    @pl.when(jnp.logical_and(l == 0, i == 0))
    def _():
        sup_ref[...] = jnp.dot(x_ref[...], W1_ref[...],
                               preferred_element_type=jnp.float32)

    @pl.when(jnp.logical_and(l == 1, i == 0))
    def _():
        sup_ref[...] = jnp.dot(jnp.maximum(h1_ref[...], 0.0), W2_ref[...],
                               preferred_element_type=jnp.float32)

    @pl.when(jnp.logical_and(l == 2, i == 0))
    def _():
        sup_ref[...] = jnp.dot(jnp.maximum(h2_ref[...], 0.0), W3_ref[...],
                               preferred_element_type=jnp.float32)

    h = jnp.dot(adj_ref[...], sup_ref[...], preferred_element_type=jnp.float32)

    @pl.when(l == 0)
    def _():
        h1_ref[pl.ds(i * _BM, _BM), :] = h + b1_ref[...]

    @pl.when(l == 1)
    def _():
        h2_ref[pl.ds(i * _BM, _BM), :] = h + b2_ref[...]

    @pl.when(l == 2)
    def _():
        h3 = h + b3_ref[...]
        r1 = jnp.maximum(h1_ref[pl.ds(i * _BM, _BM), :], 0.0)
        r2 = jnp.maximum(h2_ref[pl.ds(i * _BM, _BM), :], 0.0)
        r3 = jnp.maximum(h3, 0.0)
        out = (jnp.dot(r1, Wf_ref[0:_H, :], preferred_element_type=jnp.float32)
               + jnp.dot(r2, Wf_ref[_H:2 * _H, :],
                         preferred_element_type=jnp.float32)
               + jnp.dot(r3, Wf_ref[2 * _H:3 * _H, :],
                         preferred_element_type=jnp.float32)
               + bf_ref[...])
        m = jnp.max(out, axis=1, keepdims=True)
        e = jnp.exp(out - m)
        s = jnp.sum(e, axis=1, keepdims=True)
        logp_ref[...] = out - m - jnp.log(s)
        p_ref[...] = e / s


def kernel(x, adj, W1, b1, W2, b2, W3, b3, Wf, bf):
    _const = lambda bs: pl.BlockSpec(bs, lambda l, i: (0, 0))
    outs = pl.pallas_call(
        _jknet_body,
        grid=(3, _G),
        in_specs=[
            _const((_N, _D)),
            pl.BlockSpec((_BM, _N), lambda l, i: (i, 0)),
            _const((_D, _H)), _const((1, _H)),
            _const((_H, _H)), _const((1, _H)),
            _const((_H, _H)), _const((1, _H)),
            _const((3 * _H, _C)), _const((1, _C)),
        ],
        out_specs=[
            pl.BlockSpec((_BM, _C), lambda l, i: (i, 0)),
            pl.BlockSpec((_BM, _C), lambda l, i: (i, 0)),
        ],
        out_shape=[
            jax.ShapeDtypeStruct((_N, _C), jnp.float32),
            jax.ShapeDtypeStruct((_N, _C), jnp.float32),
        ],
        scratch_shapes=[
            pltpu.VMEM((_N, _H), jnp.float32),
            pltpu.VMEM((_N, _H), jnp.float32),
            pltpu.VMEM((_N, _H), jnp.float32),
        ],
        compiler_params=pltpu.CompilerParams(
            dimension_semantics=("arbitrary", "arbitrary")),
    )(x, adj, W1, b1.reshape(1, _H), W2, b2.reshape(1, _H),
      W3, b3.reshape(1, _H), Wf, bf.reshape(1, _C))
    return (outs[0], outs[1])


# speedup vs baseline: 1.0950x; 1.0950x over previous
import jax
import jax.numpy as jnp
from jax.experimental import pallas as pl
from jax.experimental.pallas import tpu as pltpu

_N = 10000
_D = 128
_H = 128
_C = 64
_BM = 400
_G = _N // _BM


def _layer1_body(x_ref, adj_ref, W1_ref, b1_ref,
                 h1_ref, adjbf_ref, sup_ref):
    i = pl.program_id(0)

    @pl.when(i == 0)
    def _():
        sup_ref[...] = jnp.dot(
            x_ref[...], W1_ref[...],
            preferred_element_type=jnp.float32).astype(jnp.bfloat16)

    abf = adj_ref[...].astype(jnp.bfloat16)
    adjbf_ref[...] = abf
    h1_ref[...] = jnp.dot(abf, sup_ref[...],
                          preferred_element_type=jnp.float32) + b1_ref[...]


def _layer23_body(h1_ref, adjbf_ref, W2_ref, b2_ref, W3_ref, b3_ref,
                  Wf_ref, bf_ref, logp_ref, p_ref, h2_ref, sup_ref):
    l = pl.program_id(0)
    i = pl.program_id(1)

    @pl.when(jnp.logical_and(l == 0, i == 0))
    def _():
        sup_ref[...] = jnp.dot(
            jnp.maximum(h1_ref[...], 0.0), W2_ref[...],
            preferred_element_type=jnp.float32).astype(jnp.bfloat16)

    @pl.when(jnp.logical_and(l == 1, i == 0))
    def _():
        sup_ref[...] = jnp.dot(
            jnp.maximum(h2_ref[...], 0.0), W3_ref[...],
            preferred_element_type=jnp.float32).astype(jnp.bfloat16)

    h = jnp.dot(adjbf_ref[...], sup_ref[...],
                preferred_element_type=jnp.float32)

    @pl.when(l == 0)
    def _():
        h2_ref[pl.ds(i * _BM, _BM), :] = h + b2_ref[...]

    @pl.when(l == 1)
    def _():
        h3 = h + b3_ref[...]
        r1 = jnp.maximum(h1_ref[pl.ds(i * _BM, _BM), :], 0.0)
        r2 = jnp.maximum(h2_ref[pl.ds(i * _BM, _BM), :], 0.0)
        r3 = jnp.maximum(h3, 0.0)
        out = (jnp.dot(r1, Wf_ref[0:_H, :], preferred_element_type=jnp.float32)
               + jnp.dot(r2, Wf_ref[_H:2 * _H, :],
                         preferred_element_type=jnp.float32)
               + jnp.dot(r3, Wf_ref[2 * _H:3 * _H, :],
                         preferred_element_type=jnp.float32)
               + bf_ref[...])
        m = jnp.max(out, axis=1, keepdims=True)
        e = jnp.exp(out - m)
        s = jnp.sum(e, axis=1, keepdims=True)
        logp_ref[...] = out - m - jnp.log(s)
        p_ref[...] = e / s


def kernel(x, adj, W1, b1, W2, b2, W3, b3, Wf, bf):
    seq = ("arbitrary", "arbitrary")

    h1, adj_bf = pl.pallas_call(
        _layer1_body,
        grid=(_G,),
        in_specs=[
            pl.BlockSpec((_N, _D), lambda i: (0, 0)),
            pl.BlockSpec((_BM, _N), lambda i: (i, 0)),
            pl.BlockSpec((_D, _H), lambda i: (0, 0)),
            pl.BlockSpec((1, _H), lambda i: (0, 0)),
        ],
        out_specs=[
            pl.BlockSpec((_BM, _H), lambda i: (i, 0)),
            pl.BlockSpec((_BM, _N), lambda i: (i, 0)),
        ],
        out_shape=[
            jax.ShapeDtypeStruct((_N, _H), jnp.float32),
            jax.ShapeDtypeStruct((_N, _N), jnp.bfloat16),
        ],
        scratch_shapes=[pltpu.VMEM((_N, _H), jnp.bfloat16)],
        compiler_params=pltpu.CompilerParams(
            dimension_semantics=("arbitrary",)),
    )(x, adj, W1, b1.reshape(1, _H))

    _const = lambda bs: pl.BlockSpec(bs, lambda l, i: (0, 0))
    outs = pl.pallas_call(
        _layer23_body,
        grid=(2, _G),
        in_specs=[
            _const((_N, _H)),
            pl.BlockSpec((_BM, _N), lambda l, i: (i, 0)),
            _const((_H, _H)), _const((1, _H)),
            _const((_H, _H)), _const((1, _H)),
            _const((3 * _H, _C)), _const((1, _C)),
        ],
        out_specs=[
            pl.BlockSpec((_BM, _C), lambda l, i: (i, 0)),
            pl.BlockSpec((_BM, _C), lambda l, i: (i, 0)),
        ],
        out_shape=[
            jax.ShapeDtypeStruct((_N, _C), jnp.float32),
            jax.ShapeDtypeStruct((_N, _C), jnp.float32),
        ],
        scratch_shapes=[
            pltpu.VMEM((_N, _H), jnp.float32),
            pltpu.VMEM((_N, _H), jnp.bfloat16),
        ],
        compiler_params=pltpu.CompilerParams(dimension_semantics=seq),
    )(h1, adj_bf, W2, b2.reshape(1, _H), W3, b3.reshape(1, _H),
      Wf, bf.reshape(1, _C))
    return (outs[0], outs[1])
